# baseline (speedup 1.0000x reference)
import jax
import jax.numpy as jnp
from jax import lax
from jax.experimental import pallas as pl
from jax.experimental.pallas import tpu as pltpu
from jax.experimental.pallas import tpu_sc as plsc

_VOCAB = 1000000
_DIM = 64
_L = 16

_INFO = plsc.get_sparse_core_info()
_NC = _INFO.num_cores
_NS = _INFO.num_subcores
_NW = _NC * _NS

_ROWS = 16384
_B_TOTAL = _ROWS * 50
_B_PER_W = _B_TOTAL // _NW
_CHUNK = 400
_OROWS = _CHUNK // 50
_GRP = 80
_NG = _CHUNK // _GRP
_ITERS = _B_PER_W // _CHUNK


def _sc_body(tab_hbm, idx_hbm, out_hbm,
             idx_v, g_v, emb0_v, emb1_v, sem_i, sem_g, sem_o):
    wid = lax.axis_index("s") * _NC + lax.axis_index("c")
    base = wid * _B_PER_W
    embs = (emb0_v, emb1_v)

    def load_idx(t):
        pltpu.sync_copy(idx_hbm.at[pl.ds(base + t * _CHUNK, _CHUNK)], idx_v)
        for i in range(_CHUNK // _L):
            v = idx_v[pl.ds(i * _L, _L)]
            g = jnp.where(v == 0, _VOCAB - 1, v - 1)
            r, c = divmod(i * _L, _GRP)
            g_v[r, pl.ds(c, _L)] = g

    def fire_gathers(buf):
        return [pltpu.async_copy(
            tab_hbm.at[g_v.at[r]], buf.at[pl.ds(r * _GRP, _GRP)], sem_g)
            for r in range(_NG)]

    def fire_out(t, buf):
        return [pltpu.async_copy(
            buf, out_hbm.at[pl.ds(base + t * _CHUNK, _CHUNK)], sem_o)]

    load_idx(0)
    for h in fire_gathers(embs[0]):
        h.wait()

    def chunk_body(t, carry):
        b_cur = lax.rem(t, 2)

        def do(parity):
            cur = embs[parity]
            nxt = embs[1 - parity]
            ohs = fire_out(t, cur)
            load_idx(t + 1)
            ghs = fire_gathers(nxt)
            for h in ohs:
                h.wait()
            for h in ghs:
                h.wait()

        @pl.when(b_cur == 0)
        def _():
            do(0)

        @pl.when(b_cur == 1)
        def _():
            do(1)

        return carry

    lax.fori_loop(0, _ITERS - 1, chunk_body, 0, unroll=False)

    last = (_ITERS - 1) % 2
    for h in fire_out(_ITERS - 1, embs[last]):
        h.wait()


@jax.jit
def _sc_gather(tab, idx_flat):
    mesh = plsc.VectorSubcoreMesh(core_axis_name="c", subcore_axis_name="s")
    fn = pl.kernel(
        _sc_body,
        mesh=mesh,
        compiler_params=pltpu.CompilerParams(use_tc_tiling_on_sc=False),
        out_type=jax.ShapeDtypeStruct((_B_TOTAL, _DIM), jnp.float32),
        scratch_types=[
            pltpu.VMEM((_CHUNK,), jnp.int32),
            pltpu.VMEM((_NG, _GRP), jnp.int32),
            pltpu.VMEM((_CHUNK, _DIM), jnp.float32),
            pltpu.VMEM((_CHUNK, _DIM), jnp.float32),
            pltpu.SemaphoreType.DMA,
            pltpu.SemaphoreType.DMA,
            pltpu.SemaphoreType.DMA,
        ],
    )
    return fn(tab, idx_flat)


def kernel(idx, embedding, nan_mask, impute_values, missing_vector):
    idx_flat = idx.reshape(-1).astype(jnp.int32)
    emb_t = embedding.T
    msk_t = nan_mask.T
    cols = lax.broadcasted_iota(jnp.int32, (1, _VOCAB), 1)
    tab_t = jnp.where(msk_t, impute_values.astype(jnp.float32)[:, None],
                      emb_t)
    tab_t = jnp.where(cols == _VOCAB - 1,
                      missing_vector.reshape(1, _DIM).T, tab_t)
    tab_t = lax.optimization_barrier(tab_t)
    out = _sc_gather(tab_t.T, idx_flat)
    return out.reshape(idx.shape + (_DIM,))
